# baseline (device time: 218631 ns/iter reference)
import jax
import jax.numpy as jnp
from jax import lax
from jax.experimental import pallas as pl
from jax.experimental.pallas import tpu as pltpu

N_DEV = 16
M = 4096
N = 2048
CHUNK = M // N_DEV
HALF = N // 2
SUB = 2
SUBW = HALF // SUB
DEPTH = 4
HOPS = 2 * (N_DEV - 1)


def kernel(x, w_mat):
    def body(x_ref, w_ref, out_ref, *scratch):
        comm = {}
        send_sems = {}
        recv_sems = {}
        it = iter(scratch)
        for d in ("r", "l"):
            for s in range(SUB):
                comm[(d, s)] = next(it)
                send_sems[(d, s)] = next(it)
                recv_sems[(d, s)] = next(it)

        my = lax.axis_index("i")
        left = lax.rem(my + N_DEV - 1, N_DEV)
        right = lax.rem(my + 1, N_DEV)
        dir_target = {"r": right, "l": left}

        barrier_sem = pltpu.get_barrier_semaphore()
        for nbr in (left, right):
            pl.semaphore_signal(
                barrier_sem, inc=1,
                device_id=(nbr,), device_id_type=pl.DeviceIdType.MESH,
            )
        pl.semaphore_wait(barrier_sem, 2)

        col_base = {"r": 0, "l": HALF}

        def rows(c):
            return pl.ds(c * CHUNK, CHUNK)

        out_ref[...] = jnp.dot(
            x_ref[...].astype(jnp.bfloat16),
            w_ref[...].astype(jnp.bfloat16),
            preferred_element_type=jnp.float32,
        )

        def partial(c, d):
            return out_ref[rows(c), pl.ds(col_base[d], HALF)].astype(
                jnp.bfloat16)

        def reduce_chunk(d, h):
            off = N_DEV - h - 1 if d == "r" else h + 1
            return lax.rem(my + off, N_DEV)

        def gather_chunk(d, t):
            off = N_DEV - t if d == "r" else t
            return lax.rem(my + off, N_DEV)

        def send_desc(d, s, h):
            return pltpu.make_async_remote_copy(
                src_ref=comm[(d, s)].at[h % DEPTH],
                dst_ref=comm[(d, s)].at[(h + 1) % DEPTH],
                send_sem=send_sems[(d, s)].at[h % DEPTH],
                recv_sem=recv_sems[(d, s)].at[(h + 1) % DEPTH],
                device_id=(dir_target[d],),
                device_id_type=pl.DeviceIdType.MESH,
            )

        sent = {}
        for d in ("r", "l"):
            p = partial(my, d)
            for s in range(SUB):
                comm[(d, s)][0, :, :] = p[:, s * SUBW:(s + 1) * SUBW]
        for d in ("r", "l"):
            for s in range(SUB):
                rdma = send_desc(d, s, 0)
                rdma.start()
                sent[(d, s)] = [rdma]

        for h in range(HOPS):
            reduce_hop = h < N_DEV - 1
            p_bf = {}
            if reduce_hop:
                for d in ("r", "l"):
                    p_bf[d] = partial(reduce_chunk(d, h), d)

            for s in range(SUB):
                for d in ("r", "l"):
                    if h >= DEPTH - 1:
                        sent[(d, s)].pop(0).wait_send()
                    recv = send_desc(d, s, h)
                    recv.wait_recv()
                    slot = (h + 1) % DEPTH
                    cols = pl.ds(col_base[d] + s * SUBW, SUBW)
                    if reduce_hop:
                        summed = (
                            comm[(d, s)][slot, :, :]
                            + p_bf[d][:, s * SUBW:(s + 1) * SUBW]
                        )
                        comm[(d, s)][slot, :, :] = summed
                        if h < HOPS - 1:
                            nxt = send_desc(d, s, h + 1)
                            nxt.start()
                            sent[(d, s)].append(nxt)
                        if h == N_DEV - 2:
                            out_ref[rows(reduce_chunk(d, h)), cols] = (
                                summed.astype(jnp.float32))
                    else:
                        if h < HOPS - 1:
                            nxt = send_desc(d, s, h + 1)
                            nxt.start()
                            sent[(d, s)].append(nxt)
                        t = h - (N_DEV - 1)
                        c = gather_chunk(d, t)
                        out_ref[rows(c), cols] = comm[(d, s)][
                            slot, :, :].astype(jnp.float32)

        for d in ("r", "l"):
            for s in range(SUB):
                for rdma in sent[(d, s)]:
                    rdma.wait_send()

    scratch_shapes = []
    for _d in ("r", "l"):
        for _s in range(SUB):
            scratch_shapes.append(
                pltpu.VMEM((DEPTH, CHUNK, SUBW), jnp.bfloat16))
            scratch_shapes.append(pltpu.SemaphoreType.DMA((DEPTH,)))
            scratch_shapes.append(pltpu.SemaphoreType.DMA((DEPTH,)))

    return pl.pallas_call(
        body,
        out_shape=jax.ShapeDtypeStruct((M, N), jnp.float32),
        in_specs=[
            pl.BlockSpec(memory_space=pltpu.VMEM),
            pl.BlockSpec(memory_space=pltpu.VMEM),
        ],
        out_specs=pl.BlockSpec(memory_space=pltpu.VMEM),
        scratch_shapes=scratch_shapes,
        compiler_params=pltpu.CompilerParams(
            collective_id=0, vmem_limit_bytes=100 * 1024 * 1024
        ),
    )(x, w_mat)


# device time: 218610 ns/iter; 1.0001x vs baseline; 1.0001x over previous
import jax
import jax.numpy as jnp
from jax import lax
from jax.experimental import pallas as pl
from jax.experimental.pallas import tpu as pltpu

N_DEV = 16
M = 4096
N = 2048
CHUNK = M // N_DEV
HALF = N // 2
SUB = 2
SUBW = HALF // SUB
DEPTH = 4
HOPS = 2 * (N_DEV - 1)

RING = [0, 1, 5, 9, 13, 14, 10, 6, 2, 3, 7, 11, 15, 12, 8, 4]
RING_POS = [0] * N_DEV
for _p, _dev in enumerate(RING):
    RING_POS[_dev] = _p


def kernel(x, w_mat):
    idx = lax.axis_index("i")
    k = jnp.take(jnp.array(RING_POS, jnp.int32), idx)
    rt = jnp.take(jnp.array(RING, jnp.int32), lax.rem(k + 1, N_DEV))
    lf = jnp.take(jnp.array(RING, jnp.int32), lax.rem(k + N_DEV - 1, N_DEV))
    k, rt, lf = (v.reshape(1) for v in (k, rt, lf))

    def body(x_ref, w_ref, k_ref, rt_ref, lf_ref, out_ref, *scratch):
        comm = {}
        send_sems = {}
        recv_sems = {}
        it = iter(scratch)
        for d in ("r", "l"):
            for s in range(SUB):
                comm[(d, s)] = next(it)
                send_sems[(d, s)] = next(it)
                recv_sems[(d, s)] = next(it)

        my = k_ref[0]
        right = rt_ref[0]
        left = lf_ref[0]
        dir_target = {"r": right, "l": left}

        barrier_sem = pltpu.get_barrier_semaphore()
        for nbr in (left, right):
            pl.semaphore_signal(
                barrier_sem, inc=1,
                device_id=(nbr,), device_id_type=pl.DeviceIdType.MESH,
            )
        pl.semaphore_wait(barrier_sem, 2)

        col_base = {"r": 0, "l": HALF}

        def rows(c):
            return pl.ds(c * CHUNK, CHUNK)

        out_ref[...] = jnp.dot(
            x_ref[...].astype(jnp.bfloat16),
            w_ref[...].astype(jnp.bfloat16),
            preferred_element_type=jnp.float32,
        )

        def partial(c, d):
            return out_ref[rows(c), pl.ds(col_base[d], HALF)].astype(
                jnp.bfloat16)

        def reduce_chunk(d, h):
            off = N_DEV - h - 1 if d == "r" else h + 1
            return lax.rem(my + off, N_DEV)

        def gather_chunk(d, t):
            off = N_DEV - t if d == "r" else t
            return lax.rem(my + off, N_DEV)

        def send_desc(d, s, h):
            return pltpu.make_async_remote_copy(
                src_ref=comm[(d, s)].at[h % DEPTH],
                dst_ref=comm[(d, s)].at[(h + 1) % DEPTH],
                send_sem=send_sems[(d, s)].at[h % DEPTH],
                recv_sem=recv_sems[(d, s)].at[(h + 1) % DEPTH],
                device_id=(dir_target[d],),
                device_id_type=pl.DeviceIdType.MESH,
            )

        sent = {}
        for d in ("r", "l"):
            p = partial(my, d)
            for s in range(SUB):
                comm[(d, s)][0, :, :] = p[:, s * SUBW:(s + 1) * SUBW]
        for d in ("r", "l"):
            for s in range(SUB):
                rdma = send_desc(d, s, 0)
                rdma.start()
                sent[(d, s)] = [rdma]

        for h in range(HOPS):
            reduce_hop = h < N_DEV - 1
            p_bf = {}
            if reduce_hop:
                for d in ("r", "l"):
                    p_bf[d] = partial(reduce_chunk(d, h), d)

            for s in range(SUB):
                for d in ("r", "l"):
                    if h >= DEPTH - 1:
                        sent[(d, s)].pop(0).wait_send()
                    recv = send_desc(d, s, h)
                    recv.wait_recv()
                    slot = (h + 1) % DEPTH
                    cols = pl.ds(col_base[d] + s * SUBW, SUBW)
                    if reduce_hop:
                        summed = (
                            comm[(d, s)][slot, :, :]
                            + p_bf[d][:, s * SUBW:(s + 1) * SUBW]
                        )
                        comm[(d, s)][slot, :, :] = summed
                        if h < HOPS - 1:
                            nxt = send_desc(d, s, h + 1)
                            nxt.start()
                            sent[(d, s)].append(nxt)
                        if h == N_DEV - 2:
                            out_ref[rows(reduce_chunk(d, h)), cols] = (
                                summed.astype(jnp.float32))
                    else:
                        if h < HOPS - 1:
                            nxt = send_desc(d, s, h + 1)
                            nxt.start()
                            sent[(d, s)].append(nxt)
                        t = h - (N_DEV - 1)
                        c = gather_chunk(d, t)
                        out_ref[rows(c), cols] = comm[(d, s)][
                            slot, :, :].astype(jnp.float32)

        for d in ("r", "l"):
            for s in range(SUB):
                for rdma in sent[(d, s)]:
                    rdma.wait_send()

    scratch_shapes = []
    for _d in ("r", "l"):
        for _s in range(SUB):
            scratch_shapes.append(
                pltpu.VMEM((DEPTH, CHUNK, SUBW), jnp.bfloat16))
            scratch_shapes.append(pltpu.SemaphoreType.DMA((DEPTH,)))
            scratch_shapes.append(pltpu.SemaphoreType.DMA((DEPTH,)))

    return pl.pallas_call(
        body,
        out_shape=jax.ShapeDtypeStruct((M, N), jnp.float32),
        in_specs=[
            pl.BlockSpec(memory_space=pltpu.VMEM),
            pl.BlockSpec(memory_space=pltpu.VMEM),
            pl.BlockSpec(memory_space=pltpu.SMEM),
            pl.BlockSpec(memory_space=pltpu.SMEM),
            pl.BlockSpec(memory_space=pltpu.SMEM),
        ],
        out_specs=pl.BlockSpec(memory_space=pltpu.VMEM),
        scratch_shapes=scratch_shapes,
        compiler_params=pltpu.CompilerParams(
            collective_id=0, vmem_limit_bytes=100 * 1024 * 1024
        ),
    )(x, w_mat, k, rt, lf)


# device time: 214054 ns/iter; 1.0214x vs baseline; 1.0213x over previous
import jax
import jax.numpy as jnp
from jax import lax
from jax.experimental import pallas as pl
from jax.experimental.pallas import tpu as pltpu

N_DEV = 16
M = 4096
N = 2048
CHUNK = M // N_DEV
HALF = N // 2
SUB = 2
SUBW = HALF // SUB
DEPTH = 6
HOPS = 2 * (N_DEV - 1)

RING = [0, 1, 5, 9, 13, 14, 10, 6, 2, 3, 7, 11, 15, 12, 8, 4]
RING_POS = [0] * N_DEV
for _p, _dev in enumerate(RING):
    RING_POS[_dev] = _p


def kernel(x, w_mat):
    idx = lax.axis_index("i")
    k = jnp.take(jnp.array(RING_POS, jnp.int32), idx)
    rt = jnp.take(jnp.array(RING, jnp.int32), lax.rem(k + 1, N_DEV))
    lf = jnp.take(jnp.array(RING, jnp.int32), lax.rem(k + N_DEV - 1, N_DEV))
    k, rt, lf = (v.reshape(1) for v in (k, rt, lf))

    def body(x_ref, w_ref, k_ref, rt_ref, lf_ref, out_ref, *scratch):
        comm = {}
        send_sems = {}
        recv_sems = {}
        it = iter(scratch)
        for d in ("r", "l"):
            for s in range(SUB):
                comm[(d, s)] = next(it)
                send_sems[(d, s)] = next(it)
                recv_sems[(d, s)] = next(it)

        my = k_ref[0]
        right = rt_ref[0]
        left = lf_ref[0]
        dir_target = {"r": right, "l": left}

        barrier_sem = pltpu.get_barrier_semaphore()
        for nbr in (left, right):
            pl.semaphore_signal(
                barrier_sem, inc=1,
                device_id=(nbr,), device_id_type=pl.DeviceIdType.MESH,
            )
        pl.semaphore_wait(barrier_sem, 2)

        w_b = {
            "r": w_ref[:, 0:HALF].astype(jnp.bfloat16),
            "l": w_ref[:, HALF:N].astype(jnp.bfloat16),
        }
        col_base = {"r": 0, "l": HALF}

        def rows(c):
            return pl.ds(c * CHUNK, CHUNK)

        def partial(c, d):
            xc = x_ref[rows(c), :].astype(jnp.bfloat16)
            return jnp.dot(xc, w_b[d], preferred_element_type=jnp.float32)

        def reduce_chunk(d, h):
            off = N_DEV - h - 1 if d == "r" else h + 1
            return lax.rem(my + off, N_DEV)

        def gather_chunk(d, t):
            off = N_DEV - t if d == "r" else t
            return lax.rem(my + off, N_DEV)

        def send_desc(d, s, h):
            return pltpu.make_async_remote_copy(
                src_ref=comm[(d, s)].at[h % DEPTH],
                dst_ref=comm[(d, s)].at[(h + 1) % DEPTH],
                send_sem=send_sems[(d, s)].at[h % DEPTH],
                recv_sem=recv_sems[(d, s)].at[(h + 1) % DEPTH],
                device_id=(dir_target[d],),
                device_id_type=pl.DeviceIdType.MESH,
            )

        sent = {}
        for d in ("r", "l"):
            p = partial(my, d).astype(jnp.bfloat16)
            for s in range(SUB):
                comm[(d, s)][0, :, :] = p[:, s * SUBW:(s + 1) * SUBW]
        for d in ("r", "l"):
            for s in range(SUB):
                rdma = send_desc(d, s, 0)
                rdma.start()
                sent[(d, s)] = [rdma]

        for h in range(HOPS):
            reduce_hop = h < N_DEV - 1
            p = {}
            if reduce_hop:
                for d in ("r", "l"):
                    p[d] = partial(reduce_chunk(d, h), d)

            for s in range(SUB):
                for d in ("r", "l"):
                    if h >= DEPTH - 1:
                        sent[(d, s)].pop(0).wait_send()
                    recv = send_desc(d, s, h)
                    recv.wait_recv()
                    slot = (h + 1) % DEPTH
                    cols = pl.ds(col_base[d] + s * SUBW, SUBW)
                    if reduce_hop:
                        summed = (
                            comm[(d, s)][slot, :, :].astype(jnp.float32)
                            + p[d][:, s * SUBW:(s + 1) * SUBW]
                        )
                        comm[(d, s)][slot, :, :] = summed.astype(jnp.bfloat16)
                        if h < HOPS - 1:
                            nxt = send_desc(d, s, h + 1)
                            nxt.start()
                            sent[(d, s)].append(nxt)
                        if h == N_DEV - 2:
                            out_ref[rows(reduce_chunk(d, h)), cols] = summed
                    else:
                        if h < HOPS - 1:
                            nxt = send_desc(d, s, h + 1)
                            nxt.start()
                            sent[(d, s)].append(nxt)
                        t = h - (N_DEV - 1)
                        c = gather_chunk(d, t)
                        out_ref[rows(c), cols] = comm[(d, s)][
                            slot, :, :].astype(jnp.float32)

        for d in ("r", "l"):
            for s in range(SUB):
                for rdma in sent[(d, s)]:
                    rdma.wait_send()

    scratch_shapes = []
    for _d in ("r", "l"):
        for _s in range(SUB):
            scratch_shapes.append(
                pltpu.VMEM((DEPTH, CHUNK, SUBW), jnp.bfloat16))
            scratch_shapes.append(pltpu.SemaphoreType.DMA((DEPTH,)))
            scratch_shapes.append(pltpu.SemaphoreType.DMA((DEPTH,)))

    return pl.pallas_call(
        body,
        out_shape=jax.ShapeDtypeStruct((M, N), jnp.float32),
        in_specs=[
            pl.BlockSpec(memory_space=pltpu.VMEM),
            pl.BlockSpec(memory_space=pltpu.VMEM),
            pl.BlockSpec(memory_space=pltpu.SMEM),
            pl.BlockSpec(memory_space=pltpu.SMEM),
            pl.BlockSpec(memory_space=pltpu.SMEM),
        ],
        out_specs=pl.BlockSpec(memory_space=pltpu.VMEM),
        scratch_shapes=scratch_shapes,
        compiler_params=pltpu.CompilerParams(
            collective_id=0, vmem_limit_bytes=100 * 1024 * 1024
        ),
    )(x, w_mat, k, rt, lf)
